# Initial kernel scaffold; baseline (speedup 1.0000x reference)
import jax
import jax.numpy as jnp
from jax.experimental import pallas as pl


def kernel(x):
    raise NotImplementedError("write your pallas kernel here")


# speedup vs baseline: 11.9080x; 11.9080x over previous
import math
import numpy as np
import jax
import jax.numpy as jnp
from jax import lax
from jax.experimental import pallas as pl
from jax.experimental.pallas import tpu as pltpu

_B = 64
_N = 196
_D = 768
_K = 4
_NB = 8


def _neighbor_table():
    n = int(math.sqrt(_N))
    loc = []
    for i in range(_N):
        ix, iy = divmod(i, n)
        wx = np.zeros(n)
        wy = np.zeros(n)
        wx[ix] = 1.0
        wy[iy] = 1.0
        for j in (1,):
            wx[(ix + j) % n] = 1.0
            wx[(ix - j) % n] = 1.0
            wy[(iy + j) % n] = 1.0
            wy[(iy - j) % n] = 1.0
        w = (wy[None, :] * wx[:, None]).reshape(-1)
        w[i] = 0.0
        loc.append(np.nonzero(w)[0])
    return np.stack(loc).astype(np.int32)


_LOCAL_NP = _neighbor_table()


def _tc_body(x_ref, li_ref, ti_ref, xl_ref):
    xb = x_ref[0]
    nrm = jnp.maximum(jnp.sqrt(jnp.sum(xb * xb, axis=1, keepdims=True)), 1e-12)
    xn = xb / nrm
    xnb = xn.astype(jnp.bfloat16)
    S = lax.dot_general(xnb, xnb, (((1,), (1,)), ((), ())),
                        preferred_element_type=jnp.float32)
    li = li_ref[...]
    colj = lax.broadcasted_iota(jnp.int32, (_N, _N), 1)
    sims = []
    for k in range(_NB):
        mk = colj == li[:, k:k + 1]
        sims.append(jnp.sum(jnp.where(mk, S, 0.0), axis=1, keepdims=True))
    sim = jnp.concatenate(sims, axis=1)

    kio = lax.broadcasted_iota(jnp.int32, (_N, _NB), 1)
    cur = sim
    top_cols = []
    nl_cols = []
    for t in range(_K):
        m = jnp.max(cur, axis=1, keepdims=True)
        cand = jnp.where(cur == m, kio, _NB)
        idx_t = jnp.min(cand, axis=1, keepdims=True)
        top_cols.append(idx_t)
        chosen = kio == idx_t
        cur = jnp.where(chosen, -jnp.inf, cur)
        nl_t = jnp.sum(jnp.where(chosen, li, 0), axis=1, keepdims=True)
        nl_cols.append(nl_t)
    ti_ref[0] = jnp.concatenate(top_cols, axis=1)

    for t in range(_K):
        g = (colj == nl_cols[t]).astype(jnp.float32)
        rows = lax.dot_general(g, xb, (((1,), (0,)), ((), ())),
                               preferred_element_type=jnp.float32,
                               precision=lax.Precision.HIGHEST)
        xl_ref[0, :, t, :] = rows


def _run_tc(x, interpret=False):
    li = jnp.asarray(_LOCAL_NP)
    return pl.pallas_call(
        _tc_body,
        grid=(_B,),
        in_specs=[
            pl.BlockSpec((1, _N, _D), lambda b: (b, 0, 0)),
            pl.BlockSpec((_N, _NB), lambda b: (0, 0)),
        ],
        out_specs=[
            pl.BlockSpec((1, _N, _K), lambda b: (b, 0, 0)),
            pl.BlockSpec((1, _N, _K, _D), lambda b: (b, 0, 0, 0)),
        ],
        out_shape=[
            jax.ShapeDtypeStruct((_B, _N, _K), jnp.int32),
            jax.ShapeDtypeStruct((_B, _N, _K, _D), jnp.float32),
        ],
        compiler_params=pltpu.CompilerParams(
            dimension_semantics=("parallel",)),
        interpret=interpret,
    )(x, li)


def kernel(x):
    ti, xl = _run_tc(x)
    return (ti.reshape(_B * _N, _K, 1), xl.reshape(_B * _N, _K, _D))
